# baseline (speedup 1.0000x reference)
import jax
import jax.numpy as jnp
from jax.experimental import pallas as pl
from jax.experimental.pallas import tpu as pltpu

FEAT = 16
HIDDEN = 128
OUT_DIM = 64
LANES = 4096
PSUB = 8
N = PSUB * LANES


def _mlp_block(ev_ref, w1_ref, w2_ref, out_ref):
    ev = ev_ref[...].reshape(FEAT + 1, N)
    xm = ev * ev[FEAT:]
    h = jax.lax.dot_general(
        w1_ref[...], xm, (((0,), (0,)), ((), ())),
        preferred_element_type=jnp.float32)
    h = jnp.maximum(h, 0.0)
    o = jax.lax.dot_general(
        w2_ref[...], h, (((0,), (0,)), ((), ())),
        preferred_element_type=jnp.float32)
    out_ref[...] = o.reshape(OUT_DIM + 1, PSUB, LANES)


@jax.jit
def kernel(events, W1, b1, W2, b2):
    B, P, F = events.shape
    ev_t = jnp.transpose(events, (2, 1, 0))
    w1_aug = jnp.zeros((FEAT + 1, HIDDEN + 1), jnp.float32)
    w1_aug = w1_aug.at[:FEAT, :HIDDEN].set(W1)
    w1_aug = w1_aug.at[FEAT, :HIDDEN].set(b1)
    w1_aug = w1_aug.at[FEAT, HIDDEN].set(1.0)
    w2_aug = jnp.zeros((HIDDEN + 1, OUT_DIM + 1), jnp.float32)
    w2_aug = w2_aug.at[:HIDDEN, :OUT_DIM].set(W2)
    w2_aug = w2_aug.at[HIDDEN, :OUT_DIM].set(b2)
    w2_aug = w2_aug.at[HIDDEN, OUT_DIM].set(1.0)
    out_t = pl.pallas_call(
        _mlp_block,
        grid=(P // PSUB, B // LANES),
        in_specs=[
            pl.BlockSpec((F, PSUB, LANES), lambda j, i: (0, j, i)),
            pl.BlockSpec((FEAT + 1, HIDDEN + 1), lambda j, i: (0, 0)),
            pl.BlockSpec((HIDDEN + 1, OUT_DIM + 1), lambda j, i: (0, 0)),
        ],
        out_specs=pl.BlockSpec((OUT_DIM + 1, PSUB, LANES), lambda j, i: (0, j, i)),
        out_shape=jax.ShapeDtypeStruct((OUT_DIM + 1, P, B), jnp.float32),
        compiler_params=pltpu.CompilerParams(
            dimension_semantics=("parallel", "parallel"),
        ),
    )(ev_t, w1_aug, w2_aug)
    return jnp.transpose(out_t, (2, 1, 0))
